# Initial kernel scaffold; baseline (speedup 1.0000x reference)
import jax
import jax.numpy as jnp
from jax.experimental import pallas as pl


def kernel(inputs, outputs, covars, wt, batch_num, in_embed_w, out_embed_w, in_rho_w, out_rho_w, covariates_w, linear_w, linear_b):
    raise NotImplementedError("write your pallas kernel here")


# speedup vs baseline: 1.7511x; 1.7511x over previous
import functools

import jax
import jax.numpy as jnp
from jax import lax
from jax.experimental import pallas as pl
from jax.experimental.pallas import tpu as pltpu
from jax.experimental.pallas import tpu_sc as plsc

_NEGS = 5
_SCALING = 0.1
_WIN = 128


def _sc_gather_all(out_idx, noise_idx, in_idx, out_embed_w, out_rho_w,
                   in_embed_w, in_rho_w):
    n_out = out_idx.shape[1]
    n_noise = noise_idx.shape[1]
    n_in = in_idx.shape[1]
    D = out_embed_w.shape[1]
    f32 = jnp.float32
    mesh = plsc.VectorSubcoreMesh(core_axis_name="c", subcore_axis_name="s")
    out_type = [
        jax.ShapeDtypeStruct((n_out, D), f32),
        jax.ShapeDtypeStruct((n_out, D), f32),
        jax.ShapeDtypeStruct((n_noise, D), f32),
        jax.ShapeDtypeStruct((n_in, D), f32),
        jax.ShapeDtypeStruct((n_in, D), f32),
    ]

    ispec = pl.BlockSpec((1, _WIN), lambda i: (0, i))
    ospec = pl.BlockSpec((_WIN, D), lambda i: (i, 0))

    @functools.partial(
        pl.kernel, out_type=out_type, mesh=mesh,
        compiler_params=pltpu.CompilerParams(use_tc_tiling_on_sc=False))
    def gk(out_idx_h, noise_idx_h, in_idx_h, oe_h, orho_h, ie_h, irho_h,
           mu_out_h, rho_out_h, noise_h, mu_in_h, rho_in_h):
        def pair_body(tab1, tab2):
            def body(i_v, o1_v, o2_v):
                pltpu.sync_copy(tab1.at[i_v.at[0]], o1_v)
                pltpu.sync_copy(tab2.at[i_v.at[0]], o2_v)
            return body

        def single_body(tab):
            def body(i_v, o_v):
                pltpu.sync_copy(tab.at[i_v.at[0]], o_v)
            return body

        pltpu.emit_pipeline(
            pair_body(oe_h, orho_h), grid=(n_out // _WIN,),
            in_specs=[ispec], out_specs=[ospec, ospec],
            core_axis_name=("c", "s"), dimension_semantics=(pltpu.PARALLEL,),
        )(out_idx_h, mu_out_h, rho_out_h)
        pltpu.emit_pipeline(
            single_body(oe_h), grid=(n_noise // _WIN,),
            in_specs=[ispec], out_specs=[ospec],
            core_axis_name=("c", "s"), dimension_semantics=(pltpu.PARALLEL,),
        )(noise_idx_h, noise_h)
        pltpu.emit_pipeline(
            pair_body(ie_h, irho_h), grid=(n_in // _WIN,),
            in_specs=[ispec], out_specs=[ospec, ospec],
            core_axis_name=("c", "s"), dimension_semantics=(pltpu.PARALLEL,),
        )(in_idx_h, mu_in_h, rho_in_h)

    return gk(out_idx, noise_idx, in_idx, out_embed_w, out_rho_w,
              in_embed_w, in_rho_w)


def _tc_math(mu_in, rho_in, eps_in, covf, covw, wT, bvec,
             mu_out, rho_out, eps_out, noise_v, B, W, D):
    GB = 128
    nblocks = B // GB
    GW = GB * W
    f32 = jnp.float32
    hi = lax.Precision.HIGHEST

    def body(mu_in_r, rho_in_r, eps_in_r, cov_r, covw_r, wT_r, b_r,
             mu_out_r, rho_out_r, eps_out_r, noise_r, kl_r, lik_r):
        @pl.when(pl.program_id(0) == 0)
        def _():
            kl_r[...] = jnp.zeros((1, 1), f32)
            lik_r[...] = jnp.zeros((1, 1), f32)

        mu_in = mu_in_r[...]
        rho_in = rho_in_r[...]
        eps_in = eps_in_r[...]
        cov = cov_r[...]
        covw = covw_r[...]
        wT = wT_r[...]
        bb = b_r[...]

        y = covw[0:1, :] + cov * (covw[1:2, :] - covw[0:1, :])
        sig_in = jnp.log(jnp.exp(rho_in) + 1.0)
        h = (jnp.dot(mu_in, wT[0:D, :], precision=hi, preferred_element_type=f32)
             + jnp.dot(y, wT[D:2 * D, :], precision=hi, preferred_element_type=f32)
             + bb)
        w_in = jnp.tanh(h) + _SCALING * sig_in * eps_in
        post_in = -0.5 * jnp.sum(eps_in * eps_in) - jnp.sum(jnp.log(sig_in))
        wsq = w_in * w_in
        prior_in = jnp.sum(jnp.log(0.5 * jnp.exp(-wsq / 2.0)
                                   + 0.5 * jnp.exp(-wsq / 0.08)))
        kl = W * (post_in - prior_in)

        rowi = lax.broadcasted_iota(jnp.int32, (GW, GB), 0) // W
        colj = lax.broadcasted_iota(jnp.int32, (GW, GB), 1)
        sel = (rowi == colj).astype(f32)
        w_inb = jnp.dot(sel, w_in, precision=hi, preferred_element_type=f32)

        mu_out = mu_out_r[...]
        rho_out = rho_out_r[...]
        eps_out = eps_out_r[...]
        sig_out = jnp.log(jnp.exp(rho_out) + 1.0)
        w_out = mu_out + _SCALING * sig_out * eps_out
        post_out = (-0.5 * jnp.sum(eps_out * eps_out)
                    - jnp.sum(jnp.log(sig_out)))
        wsq_o = w_out * w_out
        prior_out = jnp.sum(jnp.log(0.5 * jnp.exp(-wsq_o / 2.0)
                                    + 0.5 * jnp.exp(-wsq_o / 0.08)))
        kl += post_out - prior_out

        s = jnp.sum(w_inb * w_out, axis=1, keepdims=True)
        lik = jnp.sum(jnp.log(jax.nn.sigmoid(s)))
        ls = jnp.float32(0.0)
        for j in range(_NEGS):
            nj = noise_r[:, j * D:(j + 1) * D]
            sj = jnp.sum(w_inb * nj, axis=1, keepdims=True)
            ls += jnp.sum(jnp.log(jax.nn.sigmoid(-sj)))
        lik += ls / _NEGS

        kl_r[...] += kl.reshape(1, 1)
        lik_r[...] += lik.reshape(1, 1)

    acc_spec = pl.BlockSpec((1, 1), lambda i: (0, 0))
    kl_sum, lik_sum = pl.pallas_call(
        body,
        grid=(nblocks,),
        in_specs=[
            pl.BlockSpec((GB, D), lambda i: (i, 0)),
            pl.BlockSpec((GB, D), lambda i: (i, 0)),
            pl.BlockSpec((GB, D), lambda i: (i, 0)),
            pl.BlockSpec((GB, 1), lambda i: (i, 0)),
            pl.BlockSpec((2, D), lambda i: (0, 0)),
            pl.BlockSpec((2 * D, D), lambda i: (0, 0)),
            pl.BlockSpec((1, D), lambda i: (0, 0)),
            pl.BlockSpec((GW, D), lambda i: (i, 0)),
            pl.BlockSpec((GW, D), lambda i: (i, 0)),
            pl.BlockSpec((GW, D), lambda i: (i, 0)),
            pl.BlockSpec((GW, _NEGS * D), lambda i: (i, 0)),
        ],
        out_specs=[acc_spec, acc_spec],
        out_shape=[jax.ShapeDtypeStruct((1, 1), f32)] * 2,
    )(mu_in, rho_in, eps_in, covf, covw, wT, bvec,
      mu_out, rho_out, eps_out, noise_v)
    return kl_sum, lik_sum


def kernel(inputs, outputs, covars, wt, batch_num, in_embed_w, out_embed_w,
           in_rho_w, out_rho_w, covariates_w, linear_w, linear_b):
    B, W = outputs.shape
    V, D = in_embed_w.shape

    key = jax.random.key(42)
    k1, k2, k3 = jax.random.split(key, 3)
    eps_in = jax.random.normal(k1, (B, 1, D), jnp.float32).reshape(B, D)
    eps_out = jax.random.normal(k2, (B, W, D), jnp.float32).reshape(B * W, D)
    noise_idx = jax.random.randint(k3, (B * W, _NEGS), 0, V)

    out_idx = outputs.astype(jnp.int32).reshape(1, B * W)
    nz_idx = noise_idx.astype(jnp.int32).reshape(1, B * W * _NEGS)
    in_idx = inputs.astype(jnp.int32).reshape(1, B)

    mu_out_d, rho_out_d, noise_d, mu_in_d, rho_in_d = _sc_gather_all(
        out_idx, nz_idx, in_idx, out_embed_w, out_rho_w, in_embed_w, in_rho_w)

    noise_v = noise_d.reshape(B * W, _NEGS * D)
    covf = covars.astype(jnp.float32).reshape(B, 1)
    wT = linear_w.T
    bvec = linear_b.reshape(1, D)

    kl_sum, lik_sum = _tc_math(mu_in_d, rho_in_d, eps_in, covf, covariates_w,
                               wT, bvec, mu_out_d, rho_out_d, eps_out,
                               noise_v, B, W, D)
    loss = (wt[0] * kl_sum[0, 0] - lik_sum[0, 0]) / (B * W)
    return loss
